# Initial kernel scaffold; baseline (speedup 1.0000x reference)
import jax
import jax.numpy as jnp
from jax.experimental import pallas as pl


def kernel(h_V, h_E, mask_V, mask_attend, W1_w, W1_b, W2_w, W2_b, W3_w, W3_b, W11_w, W11_b, W12_w, W12_b, W13_w, W13_b, Win_w, Win_b, Wout_w, Wout_b, n1_s, n1_o, n2_s, n2_o, n3_s, n3_o, E_idx):
    raise NotImplementedError("write your pallas kernel here")


# speedup vs baseline: 1212.0540x; 1212.0540x over previous
import functools
import math

import jax
import jax.numpy as jnp
from jax import lax
from jax.experimental import pallas as pl
from jax.experimental.pallas import tpu as pltpu
from jax.experimental.pallas import tpu_sc as plsc

N = 10000
K = 16
H = 128
FF = 4 * H
NE = N * K

NC, NS = 2, 16
NW = NC * NS
RPW = NE // NW
CHUNK = 40
CPB = 5
BATCH = CHUNK * CPB
NBATCH = RPW // BATCH
NCHUNK = RPW // CHUNK

NB = 400
GRID = N // NB
EB = NB * K

_INV_SQRT2 = 1.0 / math.sqrt(2.0)


def _gelu(x):
    return 0.5 * x * (1.0 + lax.erf(x * _INV_SQRT2))


def _ln(x, s, o):
    mu = jnp.mean(x, axis=-1, keepdims=True)
    xc = x - mu
    var = jnp.mean(xc * xc, axis=-1, keepdims=True)
    return s * xc * lax.rsqrt(var + 1e-5) + o


def _dot(a, b):
    return jnp.dot(a, b, preferred_element_type=jnp.float32)


def _sc_gather_rows(table, idx3d):
    mesh = plsc.VectorSubcoreMesh(core_axis_name="c", subcore_axis_name="s")

    @functools.partial(
        pl.kernel,
        mesh=mesh,
        out_type=jax.ShapeDtypeStruct((NE, H), jnp.float32),
        scratch_types=[
            pltpu.VMEM((NCHUNK, CHUNK), jnp.int32),
            pltpu.VMEM((BATCH, H), jnp.float32),
            pltpu.VMEM((BATCH, H), jnp.float32),
            pltpu.SemaphoreType.DMA,
            pltpu.SemaphoreType.DMA,
            pltpu.SemaphoreType.DMA,
            pltpu.SemaphoreType.DMA,
        ],
    )
    def gk(table_hbm, idx_hbm, out_hbm, idx_v, buf0, buf1, gs0, gs1,
           ws0, ws1):
        wid = lax.axis_index("s") * NC + lax.axis_index("c")
        pltpu.sync_copy(idx_hbm.at[wid], idx_v)
        bufs, gsems, wsems = (buf0, buf1), (gs0, gs1), (ws0, ws1)

        def fire(t, buf, gsem):
            hs = []
            for c in range(CPB):
                h = pltpu.make_async_copy(
                    table_hbm.at[idx_v.at[t * CPB + c]],
                    buf.at[pl.ds(c * CHUNK, CHUNK)], gsem)
                h.start()
                hs.append(h)
            return hs

        gh = {0: fire(0, buf0, gs0), 1: None}
        wh = {0: None, 1: None}
        for t in range(NBATCH):
            cur, nxt = t % 2, (t + 1) % 2
            if t + 1 < NBATCH:
                if wh[nxt] is not None:
                    wh[nxt].wait()
                gh[nxt] = fire(t + 1, bufs[nxt], gsems[nxt])
            for h in gh[cur]:
                h.wait()
            w = pltpu.make_async_copy(
                bufs[cur],
                out_hbm.at[pl.ds(wid * RPW + t * BATCH, BATCH)],
                wsems[cur])
            w.start()
            wh[cur] = w
        for w in wh.values():
            if w is not None:
                w.wait()

    return gk(table, idx3d)


def _premul(hv, w):
    def body(hv_ref, w_ref, o_ref):
        o_ref[...] = _dot(hv_ref[...], w_ref[...])

    return pl.pallas_call(
        body,
        out_shape=jax.ShapeDtypeStruct((N, H), jnp.float32),
    )(hv, w)


def _node_update(hv, he, gc1, ma, mv, W1a, W1b, b1, W2, b2, W3, b3,
                 Win, bi, Wout, bo, W11c, n1s, n1o, n2s, n2o):
    def body(hv_ref, he_ref, gc_ref, ma_ref, mv_ref, W1a_ref, W1b_ref,
             b1_ref, W2_ref, b2_ref, W3_ref, b3_ref, Win_ref, bi_ref,
             Wout_ref, bo_ref, W11c_ref, n1s_ref, n1o_ref, n2s_ref,
             n2o_ref, hv2_ref, c2_ref):
        hv_ = hv_ref[...]
        ma_ = ma_ref[...]
        a1 = _dot(hv_, W1a_ref[...])
        z = _dot(he_ref[...], W1b_ref[...]) + gc_ref[...]
        z = z.reshape(NB, K, H) + a1[:, None, :] + b1_ref[...][None]
        m = _gelu(z).reshape(EB, H)
        m = _gelu(_dot(m, W2_ref[...]) + b2_ref[...])
        m = m.reshape(NB, K, H) * ma_[:, :, None]
        s = jnp.sum(m, axis=1)
        msum = jnp.sum(ma_, axis=1, keepdims=True)
        dh = (_dot(s, W3_ref[...]) + b3_ref[...] * msum) / 30.0
        v1 = _ln(hv_ + dh, n1s_ref[...], n1o_ref[...])
        f = _gelu(_dot(v1, Win_ref[...]) + bi_ref[...])
        v2 = _ln(v1 + _dot(f, Wout_ref[...]) + bo_ref[...],
                 n2s_ref[...], n2o_ref[...])
        v2 = v2 * mv_ref[...]
        hv2_ref[...] = v2
        c2_ref[...] = _dot(v2, W11c_ref[...])

    node = pl.BlockSpec((NB, H), lambda i: (i, 0))
    edge = pl.BlockSpec((EB, H), lambda i: (i, 0))
    full = lambda shp: pl.BlockSpec(shp, lambda i: (0,) * len(shp))
    return pl.pallas_call(
        body,
        grid=(GRID,),
        in_specs=[
            node, edge, edge,
            pl.BlockSpec((NB, K), lambda i: (i, 0)),
            pl.BlockSpec((NB, 1), lambda i: (i, 0)),
            full((H, H)), full((H, H)), full((1, H)),
            full((H, H)), full((1, H)),
            full((H, H)), full((1, H)),
            full((H, FF)), full((1, FF)),
            full((FF, H)), full((1, H)),
            full((H, H)),
            full((1, H)), full((1, H)), full((1, H)), full((1, H)),
        ],
        out_specs=[node, node],
        out_shape=[
            jax.ShapeDtypeStruct((N, H), jnp.float32),
            jax.ShapeDtypeStruct((N, H), jnp.float32),
        ],
        compiler_params=pltpu.CompilerParams(
            dimension_semantics=("arbitrary",)),
    )(hv, he, gc1, ma, mv, W1a, W1b, b1, W2, b2, W3, b3,
      Win, bi, Wout, bo, W11c, n1s, n1o, n2s, n2o)


def _edge_update(hv2, he, gc2, W11a, W11b, b11, W12, b12, W13, b13,
                 n3s, n3o):
    def body(hv2_ref, he_ref, gc_ref, W11a_ref, W11b_ref, b11_ref,
             W12_ref, b12_ref, W13_ref, b13_ref, n3s_ref, n3o_ref,
             heo_ref):
        he_ = he_ref[...]
        a2 = _dot(hv2_ref[...], W11a_ref[...])
        z = _dot(he_, W11b_ref[...]) + gc_ref[...]
        z = z.reshape(NB, K, H) + a2[:, None, :] + b11_ref[...][None]
        m = _gelu(z).reshape(EB, H)
        m = _gelu(_dot(m, W12_ref[...]) + b12_ref[...])
        m = _dot(m, W13_ref[...]) + b13_ref[...]
        heo_ref[...] = _ln(he_ + m, n3s_ref[...], n3o_ref[...])

    node = pl.BlockSpec((NB, H), lambda i: (i, 0))
    edge = pl.BlockSpec((EB, H), lambda i: (i, 0))
    full = lambda shp: pl.BlockSpec(shp, lambda i: (0,) * len(shp))
    return pl.pallas_call(
        body,
        grid=(GRID,),
        in_specs=[
            node, edge, edge,
            full((H, H)), full((H, H)), full((1, H)),
            full((H, H)), full((1, H)),
            full((H, H)), full((1, H)),
            full((1, H)), full((1, H)),
        ],
        out_specs=edge,
        out_shape=jax.ShapeDtypeStruct((NE, H), jnp.float32),
        compiler_params=pltpu.CompilerParams(
            dimension_semantics=("arbitrary",)),
    )(hv2, he, gc2, W11a, W11b, b11, W12, b12, W13, b13, n3s, n3o)


def kernel(h_V, h_E, mask_V, mask_attend, W1_w, W1_b, W2_w, W2_b, W3_w,
           W3_b, W11_w, W11_b, W12_w, W12_b, W13_w, W13_b, Win_w, Win_b,
           Wout_w, Wout_b, n1_s, n1_o, n2_s, n2_o, n3_s, n3_o, E_idx):
    hV = h_V.reshape(N, H)
    hE = h_E.reshape(NE, H)
    ma = mask_attend.reshape(N, K)
    mv = mask_V.reshape(N, 1)
    idx2 = E_idx.reshape(NW, NCHUNK, CHUNK)

    W1a, W1b, W1c = W1_w[:H], W1_w[H:2 * H], W1_w[2 * H:]
    W11a, W11b, W11c = W11_w[:H], W11_w[H:2 * H], W11_w[2 * H:]
    row = lambda v: v.reshape(1, -1)

    C1 = _premul(hV, W1c)
    g1 = _sc_gather_rows(C1, idx2)
    hV2, C2 = _node_update(
        hV, hE, g1, ma, mv, W1a, W1b, row(W1_b), W2_w, row(W2_b), W3_w,
        row(W3_b), Win_w, row(Win_b), Wout_w, row(Wout_b), W11c,
        row(n1_s), row(n1_o), row(n2_s), row(n2_o))
    g2 = _sc_gather_rows(C2, idx2)
    hEo = _edge_update(
        hV2, hE, g2, W11a, W11b, row(W11_b), W12_w, row(W12_b), W13_w,
        row(W13_b), row(n3_s), row(n3_o))

    return hV2.reshape(1, N, H), hEo.reshape(1, N, K, H)
